# Initial kernel scaffold; baseline (speedup 1.0000x reference)
import functools

import jax
import jax.numpy as jnp
from jax import lax
from jax.experimental import pallas as pl
from jax.experimental.pallas import tpu as pltpu
from jax.experimental.pallas import tpu_sc as plsc

N, E, D, C = 10000, 160000, 256, 16
NC, NS, L = 2, 16, 16
NP = 10240
HP = NP // NC
ROWS_T = HP // NS
ACC_R = HP + 64
ZR_T = ACC_R // NS
DW = D + L
K = 80
ECT = E // NS
NCH = ECT // K
NV = D // L

_mesh = plsc.VectorSubcoreMesh(core_axis_name="c", subcore_axis_name="s")
_f32 = jnp.float32
_i32 = jnp.int32


def _zero_rows(buf, nrows, width):
    z = jnp.zeros((L,), _f32)

    def zrow(r, _):
        for c in range(width // L):
            buf[r, pl.ds(c * L, L)] = z
        return 0

    lax.fori_loop(0, nrows, zrow, 0)


def _zero_acc(zsrc, acc_sh, sid):
    z0 = sid * ZR_T
    for off in (0, 64, 128, 192, 256):
        pltpu.sync_copy(zsrc.at[pl.ds(0, 64)], acc_sh.at[pl.ds(z0 + off, 64)])
    pltpu.sync_copy(zsrc.at[pl.ds(0, 4)], acc_sh.at[pl.ds(z0 + 320, 4)])


def _local_idx(srcv, lidxv, lo):
    for j in range(K // L):
        v = srcv[pl.ds(j * L, L)]
        own = (v >= lo) & (v < lo + HP)
        lidxv[pl.ds(j * L, L)] = jnp.where(own, v - lo, HP)


@functools.partial(
    pl.kernel,
    out_type=(
        jax.ShapeDtypeStruct((NP, D), _f32),
        jax.ShapeDtypeStruct((E,), _f32),
        jax.ShapeDtypeStruct((NP,), _f32),
        jax.ShapeDtypeStruct((NP,), _f32),
    ),
    scratch_types=[
        pltpu.VMEM((K,), _i32),
        pltpu.VMEM((K,), _i32),
        pltpu.VMEM((K,), _i32),
        pltpu.VMEM((K,), _f32),
        pltpu.VMEM((D,), _f32),
        pltpu.VMEM((K, D), _f32),
        pltpu.VMEM((K, D), _f32),
        pltpu.VMEM((K, DW), _f32),
        pltpu.VMEM((64, DW), _f32),
        pltpu.VMEM((64, D), _f32),
        pltpu.VMEM((64, D), _f32),
        pltpu.VMEM((64,), _f32),
        pltpu.VMEM((64,), _f32),
        pltpu.VMEM_SHARED((ACC_R, DW), _f32),
        pltpu.SemaphoreType.DMA,
        pltpu.SemaphoreType.DMA,
    ],
    mesh=_mesh,
)
def _pass_a(xp, srcr, tgtr, s2ir, xm1o, ewo, wsumo, dego,
            srcv, tgtv, lidxv, ewbuf, s2iv, xsv, xtv, dbuf,
            accv, xrow, xmv, wbufv, degbufv, acc_sh, sem1, sem2):
    cid = lax.axis_index("c")
    sid = lax.axis_index("s")
    lo = cid * HP
    lane = lax.iota(_i32, L)

    pltpu.sync_copy(s2ir, s2iv)
    _zero_rows(dbuf, 64, DW)
    _zero_acc(dbuf, acc_sh, sid)
    plsc.subcore_barrier()

    def chunk(k, _):
        base = sid * ECT + k * K
        pltpu.sync_copy(srcr.at[pl.ds(base, K)], srcv)
        pltpu.sync_copy(tgtr.at[pl.ds(base, K)], tgtv)
        _local_idx(srcv, lidxv, lo)
        cp1 = pltpu.async_copy(xp.at[srcv], xsv, sem1)
        cp2 = pltpu.async_copy(xp.at[tgtv], xtv, sem2)
        cp1.wait()
        cp2.wait()
        for g in range(K // L):
            def edge1(e, sv):
                acc = jnp.zeros((L,), _f32)
                for c in range(NV):
                    a = xsv[e, pl.ds(c * L, L)]
                    b = xtv[e, pl.ds(c * L, L)]
                    df = a - b
                    d2 = df * df
                    dbuf[e, pl.ds(c * L, L)] = d2
                    acc = acc + d2 * s2iv[pl.ds(c * L, L)]
                s = jnp.sum(acc)
                return jnp.where(lane == lax.rem(e, L), -s, sv)

            sv = lax.fori_loop(g * L, (g + 1) * L, edge1,
                               jnp.zeros((L,), _f32))
            ewbuf[pl.ds(g * L, L)] = jnp.exp(sv)

        def edge2(e, _):
            w = ewbuf[e]
            wv = jnp.full((L,), w)
            for c in range(NV):
                dbuf[e, pl.ds(c * L, L)] = dbuf[e, pl.ds(c * L, L)] * wv
            tail = jnp.where(lane == 0, w,
                             jnp.where(lane == 1, 1.0, 0.0))
            dbuf[e, pl.ds(D, L)] = tail
            return 0

        lax.fori_loop(0, K, edge2, 0)
        pltpu.sync_copy(dbuf, acc_sh.at[lidxv], add=True)

        @pl.when(cid == 0)
        def _():
            pltpu.sync_copy(ewbuf, ewo.at[pl.ds(base, K)])

        return 0

    lax.fori_loop(0, NCH, chunk, 0)
    plsc.subcore_barrier()

    r0 = sid * ROWS_T
    for cb in range(ROWS_T // 64):
        row = r0 + cb * 64
        pltpu.sync_copy(acc_sh.at[pl.ds(row, 64), :], accv)
        pltpu.sync_copy(xp.at[pl.ds(lo + row, 64), :], xrow)

        def rowfn(r, _):
            wsum_s = accv[r, D]
            deg_s = accv[r, D + 1]
            rdeg = 1.0 / jnp.maximum(jnp.full((L,), deg_s), 1e-30)
            for c in range(NV):
                m = jnp.exp(-accv[r, pl.ds(c * L, L)]
                            * s2iv[pl.ds(c * L, L)] * rdeg)
                xmv[r, pl.ds(c * L, L)] = xrow[r, pl.ds(c * L, L)] * m
            wbufv[r] = wsum_s
            degbufv[r] = deg_s
            return 0

        lax.fori_loop(0, 64, rowfn, 0)
        pltpu.sync_copy(xmv, xm1o.at[pl.ds(lo + row, 64), :])
        pltpu.sync_copy(wbufv, wsumo.at[pl.ds(lo + row, 64)])
        pltpu.sync_copy(degbufv, dego.at[pl.ds(lo + row, 64)])


@functools.partial(
    pl.kernel,
    out_type=jax.ShapeDtypeStruct((NP, D), _f32),
    scratch_types=[
        pltpu.VMEM((K,), _i32),
        pltpu.VMEM((K,), _i32),
        pltpu.VMEM((K,), _i32),
        pltpu.VMEM((K,), _f32),
        pltpu.VMEM((K,), _f32),
        pltpu.VMEM((K, D), _f32),
        pltpu.VMEM((K, D), _f32),
        pltpu.VMEM((HP + 64,), _f32),
        pltpu.VMEM((64, D), _f32),
        pltpu.VMEM_SHARED((ACC_R, D), _f32),
        pltpu.SemaphoreType.DMA,
    ],
    mesh=_mesh,
)
def _pass_bd(xmr, srcr, tgtr, ewr, wsumr, ho,
             srcv, tgtv, lidxv, ewv, ewnv, rows, sbuf, wsumv,
             accv, acc_sh, sem1):
    cid = lax.axis_index("c")
    sid = lax.axis_index("s")
    lo = cid * HP

    _zero_rows(sbuf, 64, D)
    _zero_acc(sbuf, acc_sh, sid)
    pltpu.sync_copy(wsumr.at[pl.ds(lo, HP)], wsumv.at[pl.ds(0, HP)])
    ones = jnp.full((L,), 1.0)
    for j in range(64 // L):
        wsumv[pl.ds(HP + j * L, L)] = ones
    plsc.subcore_barrier()

    def chunk(k, _):
        base = sid * ECT + k * K
        pltpu.sync_copy(srcr.at[pl.ds(base, K)], srcv)
        pltpu.sync_copy(tgtr.at[pl.ds(base, K)], tgtv)
        pltpu.sync_copy(ewr.at[pl.ds(base, K)], ewv)
        _local_idx(srcv, lidxv, lo)
        pltpu.async_copy(xmr.at[tgtv], rows, sem1).wait()
        for j in range(K // L):
            lv = lidxv[pl.ds(j * L, L)]
            wg = plsc.load_gather(wsumv, [lv])
            ewnv[pl.ds(j * L, L)] = ewv[pl.ds(j * L, L)] / wg

        def edge(e, _):
            w = jnp.full((L,), ewnv[e])
            for c in range(NV):
                sbuf[e, pl.ds(c * L, L)] = rows[e, pl.ds(c * L, L)] * w
            return 0

        lax.fori_loop(0, K, edge, 0)
        pltpu.sync_copy(sbuf, acc_sh.at[lidxv], add=True)
        return 0

    lax.fori_loop(0, NCH, chunk, 0)
    plsc.subcore_barrier()

    r0 = sid * ROWS_T
    for cb in range(ROWS_T // 64):
        row = r0 + cb * 64
        pltpu.sync_copy(acc_sh.at[pl.ds(row, 64), :], accv)
        pltpu.sync_copy(accv, ho.at[pl.ds(lo + row, 64), :])


@functools.partial(
    pl.kernel,
    out_type=jax.ShapeDtypeStruct((NP, D), _f32),
    scratch_types=[
        pltpu.VMEM((K,), _i32),
        pltpu.VMEM((K,), _i32),
        pltpu.VMEM((K,), _i32),
        pltpu.VMEM((K,), _f32),
        pltpu.VMEM((D,), _f32),
        pltpu.VMEM((K, D), _f32),
        pltpu.VMEM((K, D), _f32),
        pltpu.VMEM((K, D), _f32),
        pltpu.VMEM((64, D), _f32),
        pltpu.VMEM((64, D), _f32),
        pltpu.VMEM((64, D), _f32),
        pltpu.VMEM((64,), _f32),
        pltpu.VMEM_SHARED((ACC_R, D), _f32),
        pltpu.SemaphoreType.DMA,
        pltpu.SemaphoreType.DMA,
    ],
    mesh=_mesh,
)
def _pass_c(h1r, srcr, tgtr, ewr, degr, s2ir, xm2o,
            srcv, tgtv, lidxv, ewv, s2iv, xsv, xtv, dbuf,
            accv, hrow, xmv, degv, acc_sh, sem1, sem2):
    cid = lax.axis_index("c")
    sid = lax.axis_index("s")
    lo = cid * HP

    pltpu.sync_copy(s2ir, s2iv)
    _zero_rows(dbuf, 64, D)
    _zero_acc(dbuf, acc_sh, sid)
    plsc.subcore_barrier()

    def chunk(k, _):
        base = sid * ECT + k * K
        pltpu.sync_copy(srcr.at[pl.ds(base, K)], srcv)
        pltpu.sync_copy(tgtr.at[pl.ds(base, K)], tgtv)
        pltpu.sync_copy(ewr.at[pl.ds(base, K)], ewv)
        _local_idx(srcv, lidxv, lo)
        cp1 = pltpu.async_copy(h1r.at[srcv], xsv, sem1)
        cp2 = pltpu.async_copy(h1r.at[tgtv], xtv, sem2)
        cp1.wait()
        cp2.wait()

        def edge(e, _):
            w = jnp.full((L,), ewv[e])
            for c in range(NV):
                df = xsv[e, pl.ds(c * L, L)] - xtv[e, pl.ds(c * L, L)]
                dbuf[e, pl.ds(c * L, L)] = df * df * w
            return 0

        lax.fori_loop(0, K, edge, 0)
        pltpu.sync_copy(dbuf, acc_sh.at[lidxv], add=True)
        return 0

    lax.fori_loop(0, NCH, chunk, 0)
    plsc.subcore_barrier()

    r0 = sid * ROWS_T
    for cb in range(ROWS_T // 64):
        row = r0 + cb * 64
        pltpu.sync_copy(acc_sh.at[pl.ds(row, 64), :], accv)
        pltpu.sync_copy(h1r.at[pl.ds(lo + row, 64), :], hrow)
        pltpu.sync_copy(degr.at[pl.ds(lo + row, 64)], degv)

        def rowfn(r, _):
            rdeg = 1.0 / jnp.maximum(jnp.full((L,), degv[r]), 1e-30)
            for c in range(NV):
                m = jnp.exp(-accv[r, pl.ds(c * L, L)]
                            * s2iv[pl.ds(c * L, L)] * rdeg)
                xmv[r, pl.ds(c * L, L)] = hrow[r, pl.ds(c * L, L)] * m
            return 0

        lax.fori_loop(0, 64, rowfn, 0)
        pltpu.sync_copy(xmv, xm2o.at[pl.ds(lo + row, 64), :])


def _head(h2_ref, w_ref, o_ref):
    h = h2_ref[0:N, :]
    w = w_ref[...]
    logits = lax.dot_general(h, w, (((1,), (1,)), ((), ())),
                             preferred_element_type=_f32)
    m = jnp.max(logits, axis=1, keepdims=True)
    z = logits - m
    lse = jnp.log(jnp.sum(jnp.exp(z), axis=1, keepdims=True))
    o_ref[...] = z - lse


def kernel(x, edge_index, sigma, W):
    src = edge_index[0]
    tgt = edge_index[1]
    s2i = (1.0 / (sigma * sigma)).astype(_f32)
    xp = jnp.pad(x, ((0, NP - N), (0, 0)))
    xm1, ew, wsum, deg = _pass_a(xp, src, tgt, s2i)
    h1 = _pass_bd(xm1, src, tgt, ew, wsum)
    xm2 = _pass_c(h1, src, tgt, ew, deg, s2i)
    h2 = _pass_bd(xm2, src, tgt, ew, wsum)
    return pl.pallas_call(
        _head,
        out_shape=jax.ShapeDtypeStruct((N, C), _f32),
    )(h2, W)


# speedup vs baseline: 1.0819x; 1.0819x over previous
import functools

import jax
import jax.numpy as jnp
from jax import lax
from jax.experimental import pallas as pl
from jax.experimental.pallas import tpu as pltpu
from jax.experimental.pallas import tpu_sc as plsc

N, E, D, C = 10000, 160000, 256, 16
NC, NS, L = 2, 16, 16
NT = NC * NS
NP = 10240
RT = NP // NT
AR = RT + 16
EPT = E // NT
NG = EPT // L
TAIL = EPT - NG * L
EP = E + NT * NT * L + 64
K = 64
NV = D // L
SENT = NP - 1
TW = 128

_mesh = plsc.VectorSubcoreMesh(core_axis_name="c", subcore_axis_name="s")
_f32 = jnp.float32
_i32 = jnp.int32

_GDN = lax.GatherDimensionNumbers(
    offset_dims=(), collapsed_slice_dims=(0,), start_index_map=(0,))


def _shuf(v, idx):
    return lax.gather(v, idx[:, None], _GDN, (1,),
                      mode=lax.GatherScatterMode.PROMISE_IN_BOUNDS)


def _lane():
    return lax.iota(_i32, L)


def _m8(v):
    return pl.multiple_of(v, 8)


def _div_rt(v):
    return lax.shift_right_logical(v * 13108, 22)


def _hsum16(v):
    lane = _lane()
    for sh in (8, 4, 2, 1):
        v = v + _shuf(v, lane ^ sh)
    return v


def _rank16(s_incl):
    lane = _lane()
    j = jnp.zeros((L,), _i32)
    for bit in (8, 4, 2, 1):
        t = j + bit
        cnt_t = _shuf(s_incl, t - 1)
        j = jnp.where(cnt_t <= lane, t, j)
    return jnp.minimum(j, L - 1)


def _pscan16(v):
    lane = _lane()
    for sh in (1, 2, 4, 8):
        shifted = _shuf(v, jnp.maximum(lane - sh, 0))
        v = jnp.where(lane >= sh, v + shifted, v)
    return v


def _to_scalar(rv):
    out = jnp.int32(0)
    for b in range(19):
        bit = jnp.bitwise_and(lax.shift_right_logical(rv, b), 1)
        nb = lax.fori_loop(0, bit, lambda j, c: c + 1, jnp.int32(0))
        out = out + nb * (1 << b)
    return out


def _own_seg(bufref, o):
    va = bufref[pl.ds(0, L)]
    vb = bufref[pl.ds(L, L)]
    sel = jnp.where(o < L, va, vb)
    g = _shuf(sel, jnp.full((L,), jnp.bitwise_and(o, L - 1), _i32))
    bufref[pl.ds(2 * L, L)] = g
    return bufref[pl.ds(2 * L, L)][0]


def _zero_1d(buf, n):
    z = jnp.zeros((L,), _f32)

    def zg(g, _):
        buf[pl.ds(g * L, L)] = z
        return 0

    lax.fori_loop(0, n // L, zg, 0)


def _zero_2d(buf, nrows, ncols=NV):
    z = jnp.zeros((L,), _f32)

    def zrow(r, _):
        for c in range(ncols):
            buf[r, pl.ds(c * L, L)] = z
        return 0

    lax.fori_loop(0, nrows, zrow, 0)


def _sanitize_chunk(srcv, tgtv, lidxv, rel, t_seg, o0):
    lane = _lane()
    for j in range(K // L):
        valid = rel + j * L < t_seg
        sv = srcv[pl.ds(j * L, L)]
        tv = tgtv[pl.ds(j * L, L)]
        sv = jnp.where(valid, sv, SENT)
        tv = jnp.where(valid, tv, 0)
        srcv[pl.ds(j * L, L)] = sv
        tgtv[pl.ds(j * L, L)] = tv
        inr = (sv >= o0) & (sv < o0 + RT)
        lidxv[pl.ds(j * L, L)] = jnp.where(inr, sv - o0, RT)


@functools.partial(
    pl.kernel,
    out_type=jax.ShapeDtypeStruct((NT * NT,), _i32),
    scratch_types=[
        pltpu.VMEM((EPT + 24,), _i32),
        pltpu.VMEM((32,), _i32),
    ],
    mesh=_mesh,
)
def _p0a(srcr, cnto, srcall, cntbuf):
    cid = lax.axis_index("c")
    sid = lax.axis_index("s")
    p = cid * NS + sid
    lane = _lane()
    pltpu.sync_copy(srcr.at[pl.ds(_m8(p * EPT), EPT)], srcall.at[pl.ds(0, EPT)])

    def grp(g, carry):
        c0, c1 = carry
        v = _div_rt(srcall[pl.ds(g * L, L)])
        for bid in range(NT):
            pc = _hsum16(jnp.where(v == bid, 1, 0))
            if bid < L:
                c0 = jnp.where(lane == bid, c0 + pc, c0)
            else:
                c1 = jnp.where(lane == bid - L, c1 + pc, c1)
        return c0, c1

    z = jnp.zeros((L,), _i32)
    c0, c1 = lax.fori_loop(0, NG, grp, (z, z))
    vt = srcall[pl.ds(NG * L, L)]
    vt = jnp.where(lane < TAIL, _div_rt(vt), 9999)
    for bid in range(NT):
        pc = _hsum16(jnp.where(vt == bid, 1, 0))
        if bid < L:
            c0 = jnp.where(lane == bid, c0 + pc, c0)
        else:
            c1 = jnp.where(lane == bid - L, c1 + pc, c1)
    cntbuf[pl.ds(0, L)] = c0
    cntbuf[pl.ds(L, L)] = c1
    pltpu.sync_copy(cntbuf, cnto.at[pl.ds(_m8(p * NT), NT)])


@functools.partial(
    pl.kernel,
    out_type=(
        jax.ShapeDtypeStruct((EP,), _i32),
        jax.ShapeDtypeStruct((EP,), _i32),
        jax.ShapeDtypeStruct((32,), _i32),
        jax.ShapeDtypeStruct((32,), _i32),
    ),
    scratch_types=[
        pltpu.VMEM((NT * NT,), _i32),
        pltpu.VMEM((EPT + 24,), _i32),
        pltpu.VMEM((EPT + 24,), _i32),
        pltpu.VMEM((EPT + 24,), _i32),
        pltpu.VMEM((EPT + 64,), _i32),
        pltpu.VMEM((EPT + 64,), _i32),
        pltpu.VMEM((32,), _i32),
        pltpu.VMEM((16,), _i32),
    ],
    mesh=_mesh,
)
def _p0b(srcr, tgtr, cntr, ssrco, stgto, startso, t16o,
         cntv, srcall, tgtall, bktall, stg_s, stg_t, obuf, lnd):
    cid = lax.axis_index("c")
    sid = lax.axis_index("s")
    p = cid * NS + sid
    lane = _lane()
    pltpu.sync_copy(cntr, cntv)

    def rnd(g, _):
        v = cntv[pl.ds(g * L, L)]
        cntv[pl.ds(g * L, L)] = (v + (L - 1)) & ~(L - 1)
        return 0

    lax.fori_loop(0, NT * NT // L, rnd, 0)

    def csum(r, carry):
        t0, t1 = carry
        return (t0 + cntv[pl.ds(r * NT, L)],
                t1 + cntv[pl.ds(r * NT + L, L)])

    z = jnp.zeros((L,), _i32)
    t0, t1 = lax.fori_loop(0, NT, csum, (z, z))
    inc0 = _pscan16(t0)
    e0 = inc0 - t0
    tot0 = _shuf(inc0, jnp.full((L,), L - 1, _i32))
    inc1 = _pscan16(t1)
    e1 = inc1 - t1 + tot0

    def psum(r, carry):
        r0, r1 = carry
        return (r0 + cntv[pl.ds(r * NT, L)],
                r1 + cntv[pl.ds(r * NT + L, L)])

    r0, r1 = lax.fori_loop(0, p, psum, (z, z))
    off0 = e0 + r0
    off1 = e1 + r1

    @pl.when(p == 0)
    def _():
        obuf[pl.ds(0, L)] = e0
        obuf[pl.ds(L, L)] = e1
        pltpu.sync_copy(obuf, startso)

    @pl.when(p == 1)
    def _():
        obuf[pl.ds(0, L)] = t0
        obuf[pl.ds(L, L)] = t1
        pltpu.sync_copy(obuf, t16o)

    pltpu.sync_copy(srcr.at[pl.ds(_m8(p * EPT), EPT)], srcall.at[pl.ds(0, EPT)])
    pltpu.sync_copy(tgtr.at[pl.ds(_m8(p * EPT), EPT)], tgtall.at[pl.ds(0, EPT)])

    def bkt(g, _):
        bktall[pl.ds(g * L, L)] = _div_rt(srcall[pl.ds(g * L, L)])
        return 0

    lax.fori_loop(0, NG + 1, bkt, 0)

    last = jnp.full((L,), L - 1, _i32)
    for o in range(NT):
        woff = _to_scalar(off0[o] if o < L else off1[o - L])

        def body16(mi, sv, tv, carry):
            blk, pc, p_s, p_t = carry
            s_incl = _pscan16(mi)
            mcnt_v = _shuf(s_incl, last)
            lnd[pl.ds(0, L)] = mcnt_v
            mcnt = lnd[pl.ds(0, L)][0]
            inv = _rank16(s_incl)
            cs = _shuf(sv, inv)
            ct = _shuf(tv, inv)
            pcv = jnp.full((L,), pc)
            shp = jnp.maximum(lane - pcv, 0)
            ms = jnp.where(lane < pcv, p_s, _shuf(cs, shp))
            mt = jnp.where(lane < pcv, p_t, _shuf(ct, shp))
            spill = pc + mcnt >= L

            @pl.when(spill)
            def _():
                stg_s[pl.ds(_m8(blk * L), L)] = ms
                stg_t[pl.ds(_m8(blk * L), L)] = mt

            take = jnp.minimum(lane + (L - pcv), L - 1)
            n_s = jnp.where(spill, _shuf(cs, take), ms)
            n_t = jnp.where(spill, _shuf(ct, take), mt)
            spill_i = jnp.where(spill, 1, 0)
            blk = blk + spill_i
            pc = pc + mcnt - spill_i * L
            return blk, pc, n_s, n_t

        def cgrp(g, carry):
            mi = jnp.where(bktall[pl.ds(g * L, L)] == o, 1, 0)
            return body16(mi, srcall[pl.ds(g * L, L)],
                          tgtall[pl.ds(g * L, L)], carry)

        z16 = jnp.zeros((L,), _i32)
        carry = lax.fori_loop(
            0, NG, cgrp, (jnp.int32(0), jnp.int32(0), z16, z16))
        mit = jnp.where((bktall[pl.ds(NG * L, L)] == o) & (lane < TAIL), 1, 0)
        blk, pc, p_s, p_t = body16(
            mit, srcall[pl.ds(NG * L, L)], tgtall[pl.ds(NG * L, L)], carry)
        fin_s = jnp.where(lane < jnp.full((L,), pc), p_s, SENT)
        fin_t = jnp.where(lane < jnp.full((L,), pc), p_t, 0)

        @pl.when(pc > 0)
        def _():
            stg_s[pl.ds(_m8(blk * L), L)] = fin_s
            stg_t[pl.ds(_m8(blk * L), L)] = fin_t

        nblk = blk + jnp.where(pc > 0, 1, 0)

        def dma(j, _):
            pltpu.sync_copy(stg_s.at[pl.ds(_m8(j * L), L)],
                            ssrco.at[pl.ds(_m8(woff + j * L), L)])
            pltpu.sync_copy(stg_t.at[pl.ds(_m8(j * L), L)],
                            stgto.at[pl.ds(_m8(woff + j * L), L)])
            return 0

        lax.fori_loop(0, nblk, dma, 0)


@functools.partial(
    pl.kernel,
    out_type=(
        jax.ShapeDtypeStruct((NP, D), _f32),
        jax.ShapeDtypeStruct((EP,), _f32),
    ),
    scratch_types=[
        pltpu.VMEM((K,), _i32),
        pltpu.VMEM((K,), _i32),
        pltpu.VMEM((K,), _i32),
        pltpu.VMEM((K,), _f32),
        pltpu.VMEM((D,), _f32),
        pltpu.VMEM((48,), _i32),
        pltpu.VMEM((K, D), _f32),
        pltpu.VMEM((K, D), _f32),
        pltpu.VMEM((L, D), _f32),
        pltpu.VMEM((AR, D), _f32),
        pltpu.SemaphoreType.DMA,
        pltpu.SemaphoreType.DMA,
    ],
    mesh=_mesh,
)
def _pass_a1(xp, ssrcr, stgtr, startsr, t16r, s2ir,
             accro, ewo,
             srcv, tgtv, lidxv, ewbuf, s2iv, segb, xsv, xtv, dbuf,
             acc, sem1, sem2):
    cid = lax.axis_index("c")
    sid = lax.axis_index("s")
    o = cid * NS + sid
    o0 = o * RT
    lane = _lane()

    pltpu.sync_copy(s2ir, s2iv)
    pltpu.sync_copy(startsr, segb.at[pl.ds(0, 32)])
    start = _to_scalar(_own_seg(segb, o))
    pltpu.sync_copy(t16r, segb.at[pl.ds(0, 32)])
    t_seg = _own_seg(segb, o)
    _zero_2d(acc, AR)
    nch = lax.shift_right_logical(t_seg + (K - 1), 6)

    def chunk(k, _):
        base = _m8(start + k * K)
        pltpu.sync_copy(ssrcr.at[pl.ds(base, K)], srcv)
        pltpu.sync_copy(stgtr.at[pl.ds(base, K)], tgtv)
        _sanitize_chunk(srcv, tgtv, lidxv, k * K, t_seg, o0)
        cp1 = pltpu.async_copy(xp.at[srcv], xsv, sem1)
        cp2 = pltpu.async_copy(xp.at[tgtv], xtv, sem2)
        cp1.wait()
        cp2.wait()

        def group(g, _):
            lvec = lidxv[pl.ds(g * L, L)]
            sv = jnp.zeros((L,), _f32)
            for l in range(L):
                e = g * L + l
                av = jnp.zeros((L,), _f32)
                for c in range(NV):
                    df = xsv[e, pl.ds(c * L, L)] - xtv[e, pl.ds(c * L, L)]
                    d2 = df * df
                    dbuf[l, pl.ds(c * L, L)] = d2
                    av = av + d2 * s2iv[pl.ds(c * L, L)]
                s = _hsum16(av)
                sv = jnp.where(lane == l, -s, sv)
            ewg = jnp.exp(sv)
            ewbuf[pl.ds(g * L, L)] = ewg
            for l in range(L):
                wv = jnp.full((L,), ewg[l])
                r = lvec[l]
                for c in range(NV):
                    plsc.addupdate(acc.at[r, pl.ds(c * L, L)],
                                   dbuf[l, pl.ds(c * L, L)] * wv)
            return 0

        lax.fori_loop(0, K // L, group, 0)
        pltpu.sync_copy(ewbuf, ewo.at[pl.ds(base, K)])
        return 0

    lax.fori_loop(0, nch, chunk, 0)

    def epi(cb, _):
        pltpu.sync_copy(acc.at[pl.ds(_m8(cb * K), K), :],
                        accro.at[pl.ds(_m8(o0 + cb * K), K), :])
        return 0

    lax.fori_loop(0, RT // K, epi, 0)


@functools.partial(
    pl.kernel,
    out_type=(
        jax.ShapeDtypeStruct((NP, D), _f32),
        jax.ShapeDtypeStruct((NP,), _f32),
        jax.ShapeDtypeStruct((NP,), _f32),
    ),
    scratch_types=[
        pltpu.VMEM((K,), _i32),
        pltpu.VMEM((K,), _i32),
        pltpu.VMEM((K,), _f32),
        pltpu.VMEM((D,), _f32),
        pltpu.VMEM((48,), _i32),
        pltpu.VMEM((K,), _f32),
        pltpu.VMEM((K,), _f32),
        pltpu.VMEM((K, D), _f32),
        pltpu.VMEM((K, D), _f32),
        pltpu.VMEM((K, D), _f32),
        pltpu.VMEM((AR, TW), _f32),
    ],
    mesh=_mesh,
)
def _pass_a2(xp, accr, ssrcr, startsr, t16r, ewr, s2ir,
             xm1o, wsumo, dego,
             srcv, lidxv, ewv, s2iv, segb, wsbuf, dgbuf,
             xrow, xmv, accv, acc2):
    cid = lax.axis_index("c")
    sid = lax.axis_index("s")
    o = cid * NS + sid
    o0 = o * RT
    lane = _lane()

    pltpu.sync_copy(s2ir, s2iv)
    pltpu.sync_copy(startsr, segb.at[pl.ds(0, 32)])
    start = _to_scalar(_own_seg(segb, o))
    pltpu.sync_copy(t16r, segb.at[pl.ds(0, 32)])
    t_seg = _own_seg(segb, o)
    _zero_2d(acc2, AR, TW // L)
    nch = lax.shift_right_logical(t_seg + (K - 1), 6)

    def chunk(k, _):
        base = _m8(start + k * K)
        pltpu.sync_copy(ssrcr.at[pl.ds(base, K)], srcv)
        pltpu.sync_copy(ewr.at[pl.ds(base, K)], ewv)
        for j in range(K // L):
            valid = k * K + j * L < t_seg
            sv = jnp.where(valid, srcv[pl.ds(j * L, L)], SENT)
            inr = (sv >= o0) & (sv < o0 + RT)
            lidxv[pl.ds(j * L, L)] = jnp.where(inr, sv - o0, RT)

        def group(g, _):
            lvec = lidxv[pl.ds(g * L, L)]
            ewg = ewv[pl.ds(g * L, L)]
            for l in range(L):
                w = ewg[l]
                r = lvec[l]
                tail = jnp.where(lane == 0, w,
                                 jnp.where(lane == 1, 1.0, 0.0))
                plsc.addupdate(acc2.at[r, pl.ds((l % 8) * L, L)], tail)
            return 0

        lax.fori_loop(0, K // L, group, 0)
        return 0

    lax.fori_loop(0, nch, chunk, 0)

    def epi(cb, _):
        r0 = cb * K
        pltpu.sync_copy(xp.at[pl.ds(_m8(o0 + r0), K), :], xrow)
        pltpu.sync_copy(accr.at[pl.ds(_m8(o0 + r0), K), :], accv)

        def rowgrp(grp, _):
            wv = jnp.zeros((L,), _f32)
            dv = jnp.zeros((L,), _f32)
            for l in range(L):
                r = grp * L + l
                tv = acc2[r0 + r, pl.ds(0, L)]
                for sl in range(1, TW // L):
                    tv = tv + acc2[r0 + r, pl.ds(sl * L, L)]
                wv = jnp.where(lane == l, tv[0], wv)
                dv = jnp.where(lane == l, tv[1], dv)
                rdeg = 1.0 / jnp.maximum(jnp.full((L,), tv[1]), 1e-30)
                for c in range(NV):
                    m = jnp.exp(-accv[r, pl.ds(c * L, L)]
                                * s2iv[pl.ds(c * L, L)] * rdeg)
                    xmv[r, pl.ds(c * L, L)] = xrow[r, pl.ds(c * L, L)] * m
            wsbuf[pl.ds(grp * L, L)] = wv
            dgbuf[pl.ds(grp * L, L)] = dv
            return 0

        lax.fori_loop(0, K // L, rowgrp, 0)
        pltpu.sync_copy(xmv, xm1o.at[pl.ds(_m8(o0 + r0), K), :])
        pltpu.sync_copy(wsbuf, wsumo.at[pl.ds(_m8(o0 + r0), K)])
        pltpu.sync_copy(dgbuf, dego.at[pl.ds(_m8(o0 + r0), K)])
        return 0

    lax.fori_loop(0, RT // K, epi, 0)


@functools.partial(
    pl.kernel,
    out_type=jax.ShapeDtypeStruct((NP, D), _f32),
    scratch_types=[
        pltpu.VMEM((K,), _i32),
        pltpu.VMEM((K,), _i32),
        pltpu.VMEM((K,), _i32),
        pltpu.VMEM((K,), _f32),
        pltpu.VMEM((48,), _i32),
        pltpu.VMEM((K, D), _f32),
        pltpu.VMEM((AR, D), _f32),
        pltpu.VMEM((AR,), _f32),
        pltpu.SemaphoreType.DMA,
    ],
    mesh=_mesh,
)
def _pass_bd(xmr, ssrcr, stgtr, startsr, t16r, ewr, wsumr, ho,
             srcv, tgtv, lidxv, ewv, segb, rows, acc, wsumv, sem1):
    cid = lax.axis_index("c")
    sid = lax.axis_index("s")
    o = cid * NS + sid
    o0 = o * RT

    pltpu.sync_copy(startsr, segb.at[pl.ds(0, 32)])
    start = _to_scalar(_own_seg(segb, o))
    pltpu.sync_copy(t16r, segb.at[pl.ds(0, 32)])
    t_seg = _own_seg(segb, o)
    _zero_2d(acc, AR)
    pltpu.sync_copy(wsumr.at[pl.ds(_m8(o0), RT)], wsumv.at[pl.ds(0, RT)])
    wsumv[pl.ds(RT, L)] = jnp.full((L,), 1.0)
    nch = lax.shift_right_logical(t_seg + (K - 1), 6)

    def chunk(k, _):
        base = _m8(start + k * K)
        pltpu.sync_copy(ssrcr.at[pl.ds(base, K)], srcv)
        pltpu.sync_copy(stgtr.at[pl.ds(base, K)], tgtv)
        pltpu.sync_copy(ewr.at[pl.ds(base, K)], ewv)
        _sanitize_chunk(srcv, tgtv, lidxv, k * K, t_seg, o0)
        pltpu.async_copy(xmr.at[tgtv], rows, sem1).wait()

        def group(g, _):
            lvec = lidxv[pl.ds(g * L, L)]
            ewg = ewv[pl.ds(g * L, L)]
            for l in range(L):
                e = g * L + l
                w = jnp.full((L,), ewg[l])
                r = lvec[l]
                for c in range(NV):
                    plsc.addupdate(acc.at[r, pl.ds(c * L, L)],
                                   rows[e, pl.ds(c * L, L)] * w)
            return 0

        lax.fori_loop(0, K // L, group, 0)
        return 0

    lax.fori_loop(0, nch, chunk, 0)
    def epi(cb, _):
        r0 = cb * K

        def rowgrp(grp, _):
            wv = wsumv[pl.ds(r0 + grp * L, L)]
            for l in range(L):
                r = grp * L + l
                rinv = 1.0 / jnp.maximum(jnp.full((L,), wv[l]), 1e-30)
                for c in range(NV):
                    rows[r, pl.ds(c * L, L)] = (
                        acc[r0 + r, pl.ds(c * L, L)] * rinv)
            return 0

        lax.fori_loop(0, K // L, rowgrp, 0)
        pltpu.sync_copy(rows, ho.at[pl.ds(_m8(o0 + r0), K), :])
        return 0

    lax.fori_loop(0, RT // K, epi, 0)


@functools.partial(
    pl.kernel,
    out_type=jax.ShapeDtypeStruct((NP, D), _f32),
    scratch_types=[
        pltpu.VMEM((K,), _i32),
        pltpu.VMEM((K,), _i32),
        pltpu.VMEM((K,), _i32),
        pltpu.VMEM((K,), _f32),
        pltpu.VMEM((D,), _f32),
        pltpu.VMEM((48,), _i32),
        pltpu.VMEM((K, D), _f32),
        pltpu.VMEM((K, D), _f32),
        pltpu.VMEM((AR, D), _f32),
        pltpu.VMEM((AR,), _f32),
        pltpu.SemaphoreType.DMA,
        pltpu.SemaphoreType.DMA,
    ],
    mesh=_mesh,
)
def _pass_c(h1r, ssrcr, stgtr, startsr, t16r, ewr, degr, s2ir, xm2o,
            srcv, tgtv, lidxv, ewv, s2iv, segb, xsv, xtv, acc, degv,
            sem1, sem2):
    cid = lax.axis_index("c")
    sid = lax.axis_index("s")
    o = cid * NS + sid
    o0 = o * RT

    pltpu.sync_copy(s2ir, s2iv)
    pltpu.sync_copy(startsr, segb.at[pl.ds(0, 32)])
    start = _to_scalar(_own_seg(segb, o))
    pltpu.sync_copy(t16r, segb.at[pl.ds(0, 32)])
    t_seg = _own_seg(segb, o)
    _zero_2d(acc, AR)
    pltpu.sync_copy(degr.at[pl.ds(_m8(o0), RT)], degv.at[pl.ds(0, RT)])
    nch = lax.shift_right_logical(t_seg + (K - 1), 6)

    def chunk(k, _):
        base = _m8(start + k * K)
        pltpu.sync_copy(ssrcr.at[pl.ds(base, K)], srcv)
        pltpu.sync_copy(stgtr.at[pl.ds(base, K)], tgtv)
        pltpu.sync_copy(ewr.at[pl.ds(base, K)], ewv)
        _sanitize_chunk(srcv, tgtv, lidxv, k * K, t_seg, o0)
        cp1 = pltpu.async_copy(h1r.at[srcv], xsv, sem1)
        cp2 = pltpu.async_copy(h1r.at[tgtv], xtv, sem2)
        cp1.wait()
        cp2.wait()

        def group(g, _):
            lvec = lidxv[pl.ds(g * L, L)]
            ewg = ewv[pl.ds(g * L, L)]
            for l in range(L):
                e = g * L + l
                w = jnp.full((L,), ewg[l])
                r = lvec[l]
                for c in range(NV):
                    df = xsv[e, pl.ds(c * L, L)] - xtv[e, pl.ds(c * L, L)]
                    plsc.addupdate(acc.at[r, pl.ds(c * L, L)], df * df * w)
            return 0

        lax.fori_loop(0, K // L, group, 0)
        return 0

    lax.fori_loop(0, nch, chunk, 0)

    def epi(cb, _):
        r0 = cb * K
        pltpu.sync_copy(h1r.at[pl.ds(_m8(o0 + r0), K), :], xsv)

        def rowgrp(grp, _):
            dgv = degv[pl.ds(r0 + grp * L, L)]
            for l in range(L):
                r = grp * L + l
                rdeg = 1.0 / jnp.maximum(jnp.full((L,), dgv[l]), 1e-30)
                for c in range(NV):
                    m = jnp.exp(-acc[r0 + r, pl.ds(c * L, L)]
                                * s2iv[pl.ds(c * L, L)] * rdeg)
                    xtv[r, pl.ds(c * L, L)] = xsv[r, pl.ds(c * L, L)] * m
            return 0

        lax.fori_loop(0, K // L, rowgrp, 0)
        pltpu.sync_copy(xtv, xm2o.at[pl.ds(_m8(o0 + r0), K), :])
        return 0

    lax.fori_loop(0, RT // K, epi, 0)


def _head(h2_ref, w_ref, o_ref):
    h = h2_ref[0:N, :]
    w = w_ref[...]
    logits = lax.dot_general(h, w, (((1,), (1,)), ((), ())),
                             preferred_element_type=_f32)
    m = jnp.max(logits, axis=1, keepdims=True)
    z = logits - m
    lse = jnp.log(jnp.sum(jnp.exp(z), axis=1, keepdims=True))
    o_ref[...] = z - lse


def kernel(x, edge_index, sigma, W):
    src = edge_index[0]
    tgt = edge_index[1]
    s2i = (1.0 / (sigma * sigma)).astype(_f32)
    xp = jnp.pad(x, ((0, NP - N), (0, 0)))
    cnt = _p0a(src)
    ssrc, stgt, starts, t16 = _p0b(src, tgt, cnt)
    accr, ew = _pass_a1(xp, ssrc, stgt, starts, t16, s2i)
    xm1, wsum, deg = _pass_a2(xp, accr, ssrc, starts, t16, ew, s2i)
    h1 = _pass_bd(xm1, ssrc, stgt, starts, t16, ew, wsum)
    xm2 = _pass_c(h1, ssrc, stgt, starts, t16, ew, deg, s2i)
    h2 = _pass_bd(xm2, ssrc, stgt, starts, t16, ew, wsum)
    return pl.pallas_call(
        _head,
        out_shape=jax.ShapeDtypeStruct((N, C), _f32),
    )(h2, W)
